# baseline (speedup 1.0000x reference)
import functools

import jax
import jax.numpy as jnp
from jax import lax
from jax.experimental import pallas as pl
from jax.experimental.pallas import tpu as pltpu
from jax.experimental.pallas import tpu_sc as plsc

F32 = jnp.float32

_N = 10000
_E = 160000
_D = 256
_H = 32

_NC = 2
_NS = 16
_NW = _NC * _NS

_EW = 5120
_E_PAD = _EW * _NW
_GPT = _EW // 128
_N_PAD = 10240
_RPT = _N_PAD // _NS
_GARBAGE = 10016


def _worker_id():
  cid = lax.axis_index("c")
  sid = lax.axis_index("s")
  return cid, sid, cid * _NS + sid


def _sc1_body(src_i, dsts_i, v_hbm, z48, s1_out,
              idx_s, idx_d, buf0, buf1, s1_sh, semg, sems):
  cid, sid, wid = _worker_id()
  r0 = sid * _RPT
  pltpu.sync_copy(z48.at[pl.ds(r0, _RPT)], s1_sh.at[pl.ds(r0, _RPT)])
  g0 = wid * _GPT
  pltpu.sync_copy(src_i.at[pl.ds(g0, _GPT)], idx_s)
  pltpu.sync_copy(dsts_i.at[pl.ds(g0, _GPT)], idx_d)
  plsc.subcore_barrier()

  gc = 4
  nchunk = _GPT // gc
  bufs = (buf0, buf1)

  def issue_gather(c):
    b = bufs[c % 2]
    return [
        pltpu.async_copy(v_hbm.at[idx_s.at[c * gc + j]],
                         b.at[pl.ds(j * 128, 128)], semg)
        for j in range(gc)
    ]

  gd = {0: issue_gather(0)}
  sd = {}
  for c in range(nchunk):
    for d in gd[c]:
      d.wait()
    if c + 1 < nchunk:
      if c - 1 >= 0:
        for d in sd[c - 1]:
          d.wait()
      gd[c + 1] = issue_gather(c + 1)
    b = bufs[c % 2]
    sd[c] = [
        pltpu.async_copy(b.at[pl.ds(j * 128, 128)],
                         s1_sh.at[idx_d.at[c * gc + j]], sems, add=True)
        for j in range(gc)
    ]
  for c in (nchunk - 2, nchunk - 1):
    for d in sd[c]:
      d.wait()

  plsc.subcore_barrier()
  pltpu.sync_copy(s1_sh.at[pl.ds(r0, _RPT)], s1_out.at[cid, pl.ds(r0, _RPT)])


def _sc2_body(src_i, dstg_i, dsts_i, gs_hbm, gd_hbm, z32, h1_out, s2_out,
              idx_s, idx_dg, idx_ds, buf_a0, buf_a1, buf_b0, buf_b1,
              buf_h0, s2_sh, semg, semw):
  cid, sid, wid = _worker_id()
  r0 = sid * _RPT
  pltpu.sync_copy(z32.at[pl.ds(r0, _RPT)], s2_sh.at[pl.ds(r0, _RPT)])
  g0 = wid * _GPT
  pltpu.sync_copy(src_i.at[pl.ds(g0, _GPT)], idx_s)
  pltpu.sync_copy(dstg_i.at[pl.ds(g0, _GPT)], idx_dg)
  pltpu.sync_copy(dsts_i.at[pl.ds(g0, _GPT)], idx_ds)
  plsc.subcore_barrier()

  gc = 4
  ce = gc * 128
  nchunk = _GPT // gc
  e0 = wid * _EW
  bufs_a = (buf_a0, buf_a1)
  bufs_b = (buf_b0, buf_b1)

  def issue_gathers(c):
    a = bufs_a[c % 2]
    b = bufs_b[c % 2]
    out = []
    for j in range(gc):
      out.append(pltpu.async_copy(gs_hbm.at[idx_s.at[c * gc + j]],
                                  a.at[pl.ds(j * 128, 128)], semg))
      out.append(pltpu.async_copy(gd_hbm.at[idx_dg.at[c * gc + j]],
                                  b.at[pl.ds(j * 128, 128)], semg))
    return out

  gd = {0: issue_gathers(0)}
  for c in range(nchunk):
    for d in gd[c]:
      d.wait()
    if c + 1 < nchunk:
      gd[c + 1] = issue_gathers(c + 1)
    a = bufs_a[c % 2]
    b = bufs_b[c % 2]
    hb = buf_h0

    def row(q, rc):
      i0 = q * 4
      for u in range(4):
        i = i0 + u
        hb[i, pl.ds(0, 16)] = jnp.maximum(
            a[i, pl.ds(0, 16)] + b[i, pl.ds(0, 16)], 0.0)
        hb[i, pl.ds(16, 16)] = jnp.maximum(
            a[i, pl.ds(16, 16)] + b[i, pl.ds(16, 16)], 0.0)
      return rc

    lax.fori_loop(0, ce // 4, row, 0)
    pltpu.sync_copy(hb, h1_out.at[pl.ds(e0 + c * ce, ce)])
    for j in range(gc):
      pltpu.sync_copy(hb.at[pl.ds(j * 128, 128)],
                      s2_sh.at[idx_ds.at[c * gc + j]], add=True)

  plsc.subcore_barrier()
  pltpu.sync_copy(s2_sh.at[pl.ds(r0, _RPT)], s2_out.at[cid, pl.ds(r0, _RPT)])


def _sc3_body(src_i, dstg_i, ts_hbm, td_hbm, tsum_out,
              idx_s, idx_dg, buf_a0, buf_a1, buf_b0, buf_b1,
              buf_t0, buf_t1, semg, semw):
  cid, sid, wid = _worker_id()
  g0 = wid * _GPT
  pltpu.sync_copy(src_i.at[pl.ds(g0, _GPT)], idx_s)
  pltpu.sync_copy(dstg_i.at[pl.ds(g0, _GPT)], idx_dg)

  gc = 4
  ce = gc * 128
  nchunk = _GPT // gc
  e0 = wid * _EW
  bufs_a = (buf_a0, buf_a1)
  bufs_b = (buf_b0, buf_b1)
  bufs_t = (buf_t0, buf_t1)

  def issue_gathers(c):
    a = bufs_a[c % 2]
    b = bufs_b[c % 2]
    out = []
    for j in range(gc):
      out.append(pltpu.async_copy(ts_hbm.at[idx_s.at[c * gc + j]],
                                  a.at[pl.ds(j * 128, 128)], semg))
      out.append(pltpu.async_copy(td_hbm.at[idx_dg.at[c * gc + j]],
                                  b.at[pl.ds(j * 128, 128)], semg))
    return out

  gd = {0: issue_gathers(0)}
  for c in range(nchunk):
    for d in gd[c]:
      d.wait()
    if c + 1 < nchunk:
      gd[c + 1] = issue_gathers(c + 1)
    a = bufs_a[c % 2]
    b = bufs_b[c % 2]
    tb = bufs_t[c % 2]

    def row(q, rc):
      i0 = q * 4
      for u in range(4):
        i = i0 + u
        tb[i, pl.ds(0, 16)] = a[i, pl.ds(0, 16)] + b[i, pl.ds(0, 16)]
        tb[i, pl.ds(16, 16)] = a[i, pl.ds(16, 16)] + b[i, pl.ds(16, 16)]
      return rc

    lax.fori_loop(0, ce // 4, row, 0)
    pltpu.sync_copy(tb, tsum_out.at[pl.ds(e0 + c * ce, ce)])


def _tc1_body(x_ref, w_ref, col_ref, v_ref, r_ref):
  vr = jnp.dot(x_ref[...], w_ref[...], preferred_element_type=F32)
  v_ref[...] = jnp.concatenate(
      [vr[:, :_H], jnp.broadcast_to(col_ref[...], (_N, 16))], axis=1)
  r_ref[...] = vr[:, _H:]


def _tc2_body(v_ref, r_ref, s1_ref, a1_ref, b1_ref, bn1_ref, be1_ref,
              x1_ref, gs_ref, gd_ref):
  s1 = s1_ref[0, :_N, :_H] + s1_ref[1, :_N, :_H]
  rd = s1_ref[0, :_N, _H:_H + 1] + s1_ref[1, :_N, _H:_H + 1]
  clip = jnp.maximum(rd, 1.0)
  v = v_ref[:, :_H]
  r = r_ref[...]
  x1 = jnp.maximum((rd * v - s1) / clip + bn1_ref[...], 0.0)
  x1_ref[...] = x1
  gs_ref[...] = jnp.dot(x1, a1_ref[...], preferred_element_type=F32) - r
  gd_ref[...] = (jnp.dot(x1, b1_ref[...], preferred_element_type=F32) + r
                 + be1_ref[...])


def _tc3_body(x1_ref, s2_ref, s1_ref, wn2a_ref, wn2b_ref, bn2_ref,
              a2_ref, b2_ref, be2_ref, ts_ref, td_ref):
  rd = s1_ref[0, :_N, _H:_H + 1] + s1_ref[1, :_N, _H:_H + 1]
  clip = jnp.maximum(rd, 1.0)
  agg2 = (s2_ref[0, :_N, :] + s2_ref[1, :_N, :]) / clip
  x1 = x1_ref[...]
  x2 = jnp.maximum(
      jnp.dot(x1, wn2a_ref[...], preferred_element_type=F32)
      + jnp.dot(agg2, wn2b_ref[...], preferred_element_type=F32)
      + bn2_ref[...], 0.0)
  ts_ref[...] = jnp.dot(x2, a2_ref[...], preferred_element_type=F32)
  td_ref[...] = (jnp.dot(x2, b2_ref[...], preferred_element_type=F32)
                 + be2_ref[...])


def _tcm_body(h1_ref, c2_ref, hc_ref):
  hc_ref[...] = jnp.dot(h1_ref[...], c2_ref[...], preferred_element_type=F32)


def _tc4_body(hc_ref, tsum_ref, et_ref, wc_ref, bc_ref,
              pred_ref, loss_ref, sums_ref):
  i = pl.program_id(0)
  e2 = jnp.maximum(tsum_ref[...] + hc_ref[...], 0.0)
  logits = lax.dot_general(
      wc_ref[...], e2, (((1,), (1,)), ((), ())),
      preferred_element_type=F32) + bc_ref[0]
  p = 1.0 / (1.0 + jnp.exp(-logits))
  pred_ref[...] = p.reshape(pred_ref.shape)
  y = et_ref[0].astype(F32)
  pc = jnp.clip(p, 1e-7, 1.0 - 1e-7)
  bce = -(y * jnp.log(pc) + (1.0 - y) * jnp.log(1.0 - pc))
  p2 = p * p

  @pl.when(i == 0)
  def _():
    for k in range(5):
      sums_ref[k] = 0.0

  sums_ref[0] += jnp.sum(p)
  sums_ref[1] += jnp.sum(bce)
  sums_ref[2] += jnp.sum(p2)
  sums_ref[3] += jnp.sum(p2 * p)
  sums_ref[4] += jnp.sum(p2 * p2)

  @pl.when(i == pl.num_programs(0) - 1)
  def _():
    ec = float(_E)
    sp = sums_ref[0]
    m = sp / ec
    cm4 = (sums_ref[4] - 4.0 * m * sums_ref[3] + 6.0 * m * m * sums_ref[2]
           - 4.0 * m * m * m * sp) / ec + m * m * m * m
    cm4 = jnp.maximum(cm4, 0.0)
    loss_ref[0] = sums_ref[1] / ec - jnp.sqrt(jnp.sqrt(cm4)) * 0.1


def kernel(x, edge_index, edge_type, W_n1, b_n1, W_e1, b_e1, W_n2, b_n2,
           W_e2, b_e2, W_c, b_c):
  n, d = x.shape
  e = edge_index.shape[1]
  h = _H
  assert (n, e, d) == (_N, _E, _D)

  src = edge_index[0]
  dst = edge_index[1]
  pad = _E_PAD - e
  zpad = jnp.zeros((pad,), jnp.int32)
  src_g = jnp.concatenate([src, zpad]).reshape(_E_PAD // 128, 128)
  dst_g = jnp.concatenate([dst, zpad]).reshape(_E_PAD // 128, 128)
  dst_s = jnp.concatenate(
      [dst, jnp.full((pad,), _GARBAGE, jnp.int32)]).reshape(_E_PAD // 128, 128)

  z32 = jnp.zeros((_N_PAD, 32), F32)
  z48 = jnp.zeros((_N_PAD, 48), F32)
  col16 = jnp.zeros((1, 16), F32).at[0, 0].set(1.0)

  w_cat = jnp.concatenate([W_n1[d:], W_e1[2 * h:]], axis=1)
  a1, b1 = W_e1[:h], W_e1[h:2 * h]
  wn2a, wn2b = W_n2[:h], W_n2[h:]
  a2, b2, c2 = W_e2[:h], W_e2[h:2 * h], W_e2[2 * h:]
  bn1 = b_n1.reshape(1, h)
  be1 = b_e1.reshape(1, h)
  bn2 = b_n2.reshape(1, h)
  be2 = b_e2.reshape(1, h)
  eb = 16000
  nblk = e // eb
  et2 = edge_type.reshape(nblk, 1, eb)
  wc_row = W_c.reshape(1, h)

  v48, r = pl.pallas_call(
      _tc1_body,
      out_shape=[jax.ShapeDtypeStruct((n, 48), F32),
                 jax.ShapeDtypeStruct((n, h), F32)],
  )(x, w_cat, col16)

  mesh = plsc.VectorSubcoreMesh(core_axis_name="c", subcore_axis_name="s")
  sc_params = pltpu.CompilerParams(use_tc_tiling_on_sc=False)

  s1o = pl.kernel(
      _sc1_body,
      out_type=jax.ShapeDtypeStruct((_NC, _N_PAD, 48), F32),
      mesh=mesh,
      compiler_params=sc_params,
      scratch_types=[
          pltpu.VMEM((_GPT, 128), jnp.int32),
          pltpu.VMEM((_GPT, 128), jnp.int32),
          pltpu.VMEM((512, 48), F32),
          pltpu.VMEM((512, 48), F32),
          pltpu.VMEM_SHARED((_N_PAD, 48), F32),
          pltpu.SemaphoreType.DMA,
          pltpu.SemaphoreType.DMA,
      ],
  )(src_g, dst_s, v48, z48)

  x1, gsrc, gdst = pl.pallas_call(
      _tc2_body,
      out_shape=[jax.ShapeDtypeStruct((n, h), F32)] * 3,
  )(v48, r, s1o, a1, b1, bn1, be1)

  h1, s2o = pl.kernel(
      _sc2_body,
      out_type=[jax.ShapeDtypeStruct((_E_PAD, 32), F32),
                jax.ShapeDtypeStruct((_NC, _N_PAD, 32), F32)],
      mesh=mesh,
      compiler_params=sc_params,
      scratch_types=[
          pltpu.VMEM((_GPT, 128), jnp.int32),
          pltpu.VMEM((_GPT, 128), jnp.int32),
          pltpu.VMEM((_GPT, 128), jnp.int32),
          pltpu.VMEM((512, 32), F32),
          pltpu.VMEM((512, 32), F32),
          pltpu.VMEM((512, 32), F32),
          pltpu.VMEM((512, 32), F32),
          pltpu.VMEM((512, 32), F32),
          pltpu.VMEM_SHARED((_N_PAD, 32), F32),
          pltpu.SemaphoreType.DMA,
          pltpu.SemaphoreType.DMA,
      ],
  )(src_g, dst_g, dst_s, gsrc, gdst, z32)

  tsrc, tdst = pl.pallas_call(
      _tc3_body,
      out_shape=[jax.ShapeDtypeStruct((n, h), F32)] * 2,
  )(x1, s2o, s1o, wn2a, wn2b, bn2, a2, b2, be2)

  tsum = pl.kernel(
      _sc3_body,
      out_type=jax.ShapeDtypeStruct((_E_PAD, 32), F32),
      mesh=mesh,
      compiler_params=sc_params,
      scratch_types=[
          pltpu.VMEM((_GPT, 128), jnp.int32),
          pltpu.VMEM((_GPT, 128), jnp.int32),
          pltpu.VMEM((512, 32), F32),
          pltpu.VMEM((512, 32), F32),
          pltpu.VMEM((512, 32), F32),
          pltpu.VMEM((512, 32), F32),
          pltpu.VMEM((512, 32), F32),
          pltpu.VMEM((512, 32), F32),
          pltpu.SemaphoreType.DMA,
          pltpu.SemaphoreType.DMA,
      ],
  )(src_g, dst_g, tsrc, tdst)

  grid = (nblk,)
  hc = pl.pallas_call(
      _tcm_body,
      grid=grid,
      in_specs=[
          pl.BlockSpec((eb, 32), lambda i: (i, 0)),
          pl.BlockSpec((32, 32), lambda i: (0, 0)),
      ],
      out_specs=pl.BlockSpec((eb, 32), lambda i: (i, 0)),
      out_shape=jax.ShapeDtypeStruct((nblk * eb, 32), F32),
  )(h1, c2)

  pred2, loss1 = pl.pallas_call(
      _tc4_body,
      grid=grid,
      in_specs=[
          pl.BlockSpec((eb, 32), lambda i: (i, 0)),
          pl.BlockSpec((eb, 32), lambda i: (i, 0)),
          pl.BlockSpec((1, 1, eb), lambda i: (i, 0, 0)),
          pl.BlockSpec((1, 32), lambda i: (0, 0)),
          pl.BlockSpec(memory_space=pltpu.SMEM),
      ],
      out_specs=[
          pl.BlockSpec((1, 1, eb), lambda i: (i, 0, 0)),
          pl.BlockSpec(memory_space=pltpu.SMEM),
      ],
      out_shape=[jax.ShapeDtypeStruct((nblk, 1, eb), F32),
                 jax.ShapeDtypeStruct((1,), F32)],
      scratch_shapes=[pltpu.SMEM((8,), F32)],
  )(hc, tsum, et2, wc_row, b_c)

  return pred2.reshape(e), loss1[0]


# speedup vs baseline: 1.1049x; 1.1049x over previous
import functools

import jax
import jax.numpy as jnp
from jax import lax
from jax.experimental import pallas as pl
from jax.experimental.pallas import tpu as pltpu
from jax.experimental.pallas import tpu_sc as plsc

F32 = jnp.float32

_N = 10000
_E = 160000
_D = 256
_H = 32

_NC = 2
_NS = 16
_NW = _NC * _NS

_EW = 5120
_E_PAD = _EW * _NW
_GPT = _EW // 128
_N_PAD = 10240
_RPT = _N_PAD // _NS
_GARBAGE = 10016


def _worker_id():
  cid = lax.axis_index("c")
  sid = lax.axis_index("s")
  return cid, sid, cid * _NS + sid


def _sc1_body(src_i, dsts_i, v_hbm, z48, s1_out,
              idx_s, idx_d, buf0, buf1, s1_sh, semg, sems):
  cid, sid, wid = _worker_id()
  r0 = sid * _RPT
  pltpu.sync_copy(z48.at[pl.ds(r0, _RPT)], s1_sh.at[pl.ds(r0, _RPT)])
  g0 = wid * _GPT
  pltpu.sync_copy(src_i.at[pl.ds(g0, _GPT)], idx_s)
  pltpu.sync_copy(dsts_i.at[pl.ds(g0, _GPT)], idx_d)
  plsc.subcore_barrier()

  gc = 4
  nchunk = _GPT // gc
  bufs = (buf0, buf1)

  def issue_gather(c):
    b = bufs[c % 2]
    return [
        pltpu.async_copy(v_hbm.at[idx_s.at[c * gc + j]],
                         b.at[pl.ds(j * 128, 128)], semg)
        for j in range(gc)
    ]

  gd = {0: issue_gather(0)}
  sd = {}
  for c in range(nchunk):
    for d in gd[c]:
      d.wait()
    if c + 1 < nchunk:
      if c - 1 >= 0:
        for d in sd[c - 1]:
          d.wait()
      gd[c + 1] = issue_gather(c + 1)
    b = bufs[c % 2]
    sd[c] = [
        pltpu.async_copy(b.at[pl.ds(j * 128, 128)],
                         s1_sh.at[idx_d.at[c * gc + j]], sems, add=True)
        for j in range(gc)
    ]
  for c in (nchunk - 2, nchunk - 1):
    for d in sd[c]:
      d.wait()

  plsc.subcore_barrier()
  pltpu.sync_copy(s1_sh.at[pl.ds(r0, _RPT)], s1_out.at[cid, pl.ds(r0, _RPT)])


def _sc2_body(src_i, dsts_i, gs_hbm, gd_hbm, z32, h1_out, s2_out,
              idx_s, idx_ds, buf_a0, buf_a1, buf_b0, buf_b1,
              buf_h0, buf_h1, s2_sh, semg, semw, sems):
  cid, sid, wid = _worker_id()
  r0 = sid * _RPT
  pltpu.sync_copy(z32.at[pl.ds(r0, _RPT)], s2_sh.at[pl.ds(r0, _RPT)])
  g0 = wid * _GPT
  pltpu.sync_copy(src_i.at[pl.ds(g0, _GPT)], idx_s)
  pltpu.sync_copy(dsts_i.at[pl.ds(g0, _GPT)], idx_ds)
  plsc.subcore_barrier()

  gc = 4
  ce = gc * 128
  nchunk = _GPT // gc
  e0 = wid * _EW
  bufs_a = (buf_a0, buf_a1)
  bufs_b = (buf_b0, buf_b1)
  bufs_h = (buf_h0, buf_h1)

  def issue_gathers(c):
    a = bufs_a[c % 2]
    b = bufs_b[c % 2]
    out = []
    for j in range(gc):
      out.append(pltpu.async_copy(gs_hbm.at[idx_s.at[c * gc + j]],
                                  a.at[pl.ds(j * 128, 128)], semg))
      out.append(pltpu.async_copy(gd_hbm.at[idx_ds.at[c * gc + j]],
                                  b.at[pl.ds(j * 128, 128)], semg))
    return out

  gd = {0: issue_gathers(0)}
  wd = {}
  for c in range(nchunk):
    for d in gd[c]:
      d.wait()
    if c + 1 < nchunk:
      gd[c + 1] = issue_gathers(c + 1)
    if c - 1 >= 0:
      for d in wd[c - 1]:
        d.wait()
    a = bufs_a[c % 2]
    b = bufs_b[c % 2]
    hb = bufs_h[c % 2]

    def row(q, rc):
      i0 = q * 4
      for u in range(4):
        i = i0 + u
        hb[i, pl.ds(0, 16)] = jnp.maximum(
            a[i, pl.ds(0, 16)] + b[i, pl.ds(0, 16)], 0.0)
        hb[i, pl.ds(16, 16)] = jnp.maximum(
            a[i, pl.ds(16, 16)] + b[i, pl.ds(16, 16)], 0.0)
      return rc

    lax.fori_loop(0, ce // 4, row, 0)
    out = [pltpu.async_copy(hb, h1_out.at[pl.ds(e0 + c * ce, ce)], semw)]
    for j in range(gc):
      out.append(pltpu.async_copy(hb.at[pl.ds(j * 128, 128)],
                                  s2_sh.at[idx_ds.at[c * gc + j]], sems,
                                  add=True))
    wd[c] = out
  for d in wd[nchunk - 1]:
    d.wait()

  plsc.subcore_barrier()
  pltpu.sync_copy(s2_sh.at[pl.ds(r0, _RPT)], s2_out.at[cid, pl.ds(r0, _RPT)])


def _sc3_body(src_i, dstg_i, ts_hbm, td_hbm, tsum_out,
              idx_s, idx_dg, buf_a0, buf_a1, buf_b0, buf_b1,
              buf_t0, buf_t1, semg, semw):
  cid, sid, wid = _worker_id()
  g0 = wid * _GPT
  pltpu.sync_copy(src_i.at[pl.ds(g0, _GPT)], idx_s)
  pltpu.sync_copy(dstg_i.at[pl.ds(g0, _GPT)], idx_dg)

  gc = 4
  ce = gc * 128
  nchunk = _GPT // gc
  e0 = wid * _EW
  bufs_a = (buf_a0, buf_a1)
  bufs_b = (buf_b0, buf_b1)
  bufs_t = (buf_t0, buf_t1)

  def issue_gathers(c):
    a = bufs_a[c % 2]
    b = bufs_b[c % 2]
    out = []
    for j in range(gc):
      out.append(pltpu.async_copy(ts_hbm.at[idx_s.at[c * gc + j]],
                                  a.at[pl.ds(j * 128, 128)], semg))
      out.append(pltpu.async_copy(td_hbm.at[idx_dg.at[c * gc + j]],
                                  b.at[pl.ds(j * 128, 128)], semg))
    return out

  gd = {0: issue_gathers(0)}
  for c in range(nchunk):
    for d in gd[c]:
      d.wait()
    if c + 1 < nchunk:
      gd[c + 1] = issue_gathers(c + 1)
    a = bufs_a[c % 2]
    b = bufs_b[c % 2]
    tb = bufs_t[c % 2]

    def row(q, rc):
      i0 = q * 4
      for u in range(4):
        i = i0 + u
        tb[i, pl.ds(0, 16)] = a[i, pl.ds(0, 16)] + b[i, pl.ds(0, 16)]
        tb[i, pl.ds(16, 16)] = a[i, pl.ds(16, 16)] + b[i, pl.ds(16, 16)]
      return rc

    lax.fori_loop(0, ce // 4, row, 0)
    pltpu.sync_copy(tb, tsum_out.at[pl.ds(e0 + c * ce, ce)])


def _tc1_body(x_ref, w_ref, col_ref, v_ref, r_ref):
  vr = jnp.dot(x_ref[...], w_ref[...], preferred_element_type=F32)
  v_ref[...] = jnp.concatenate(
      [vr[:, :_H], jnp.broadcast_to(col_ref[...], (_N, 16))], axis=1)
  r_ref[...] = vr[:, _H:]


def _tc2_body(v_ref, r_ref, s1_ref, a1_ref, b1_ref, bn1_ref, be1_ref,
              x1_ref, gs_ref, gd_ref):
  s1 = s1_ref[0, :_N, :_H] + s1_ref[1, :_N, :_H]
  rd = s1_ref[0, :_N, _H:_H + 1] + s1_ref[1, :_N, _H:_H + 1]
  clip = jnp.maximum(rd, 1.0)
  v = v_ref[:, :_H]
  r = r_ref[...]
  x1 = jnp.maximum((rd * v - s1) / clip + bn1_ref[...], 0.0)
  x1_ref[...] = x1
  gs_ref[:_N, :] = jnp.dot(x1, a1_ref[...], preferred_element_type=F32) - r
  gs_ref[_N:, :] = jnp.zeros((_N_PAD - _N, _H), F32)
  gd_ref[:_N, :] = (jnp.dot(x1, b1_ref[...], preferred_element_type=F32) + r
                    + be1_ref[...])
  gd_ref[_N:, :] = jnp.zeros((_N_PAD - _N, _H), F32)


def _tc3_body(x1_ref, s2_ref, s1_ref, wn2a_ref, wn2b_ref, bn2_ref,
              a2_ref, b2_ref, be2_ref, ts_ref, td_ref):
  rd = s1_ref[0, :_N, _H:_H + 1] + s1_ref[1, :_N, _H:_H + 1]
  clip = jnp.maximum(rd, 1.0)
  agg2 = (s2_ref[0, :_N, :] + s2_ref[1, :_N, :]) / clip
  x1 = x1_ref[...]
  x2 = jnp.maximum(
      jnp.dot(x1, wn2a_ref[...], preferred_element_type=F32)
      + jnp.dot(agg2, wn2b_ref[...], preferred_element_type=F32)
      + bn2_ref[...], 0.0)
  ts_ref[...] = jnp.dot(x2, a2_ref[...], preferred_element_type=F32)
  td_ref[...] = (jnp.dot(x2, b2_ref[...], preferred_element_type=F32)
                 + be2_ref[...])


def _tc4_body(h1_ref, tsum_ref, et_ref, c2_ref, wc_ref, bc_ref,
              pred_ref, loss_ref, sums_ref):
  i = pl.program_id(0)
  e2 = jnp.maximum(
      tsum_ref[...] + jnp.dot(h1_ref[...], c2_ref[...],
                              preferred_element_type=F32), 0.0)
  logits = lax.dot_general(
      wc_ref[...], e2, (((1,), (1,)), ((), ())),
      preferred_element_type=F32) + bc_ref[0]
  p = 1.0 / (1.0 + jnp.exp(-logits))
  pred_ref[...] = p.reshape(pred_ref.shape)
  y = et_ref[0].astype(F32)
  pc = jnp.clip(p, 1e-7, 1.0 - 1e-7)
  bce = -(y * jnp.log(pc) + (1.0 - y) * jnp.log(1.0 - pc))
  p2 = p * p

  @pl.when(i == 0)
  def _():
    for k in range(5):
      sums_ref[k] = 0.0

  sums_ref[0] += jnp.sum(p)
  sums_ref[1] += jnp.sum(bce)
  sums_ref[2] += jnp.sum(p2)
  sums_ref[3] += jnp.sum(p2 * p)
  sums_ref[4] += jnp.sum(p2 * p2)

  @pl.when(i == pl.num_programs(0) - 1)
  def _():
    ec = float(_E)
    sp = sums_ref[0]
    m = sp / ec
    cm4 = (sums_ref[4] - 4.0 * m * sums_ref[3] + 6.0 * m * m * sums_ref[2]
           - 4.0 * m * m * m * sp) / ec + m * m * m * m
    cm4 = jnp.maximum(cm4, 0.0)
    loss_ref[0] = sums_ref[1] / ec - jnp.sqrt(jnp.sqrt(cm4)) * 0.1


def kernel(x, edge_index, edge_type, W_n1, b_n1, W_e1, b_e1, W_n2, b_n2,
           W_e2, b_e2, W_c, b_c):
  n, d = x.shape
  e = edge_index.shape[1]
  h = _H
  assert (n, e, d) == (_N, _E, _D)

  src = edge_index[0]
  dst = edge_index[1]
  pad = _E_PAD - e
  zpad = jnp.zeros((pad,), jnp.int32)
  src_g = jnp.concatenate([src, zpad]).reshape(_E_PAD // 128, 128)
  dst_g = jnp.concatenate([dst, zpad]).reshape(_E_PAD // 128, 128)
  dst_s = jnp.concatenate(
      [dst, jnp.full((pad,), _GARBAGE, jnp.int32)]).reshape(_E_PAD // 128, 128)

  z32 = jnp.zeros((_N_PAD, 32), F32)
  z48 = jnp.zeros((_N_PAD, 48), F32)
  col16 = jnp.zeros((1, 16), F32).at[0, 0].set(1.0)

  w_cat = jnp.concatenate([W_n1[d:], W_e1[2 * h:]], axis=1)
  a1, b1 = W_e1[:h], W_e1[h:2 * h]
  wn2a, wn2b = W_n2[:h], W_n2[h:]
  a2, b2, c2 = W_e2[:h], W_e2[h:2 * h], W_e2[2 * h:]
  bn1 = b_n1.reshape(1, h)
  be1 = b_e1.reshape(1, h)
  bn2 = b_n2.reshape(1, h)
  be2 = b_e2.reshape(1, h)
  eb = 16000
  nblk = e // eb
  et2 = edge_type.reshape(nblk, 1, eb)
  wc_row = W_c.reshape(1, h)

  v48, r = pl.pallas_call(
      _tc1_body,
      out_shape=[jax.ShapeDtypeStruct((n, 48), F32),
                 jax.ShapeDtypeStruct((n, h), F32)],
  )(x, w_cat, col16)

  mesh = plsc.VectorSubcoreMesh(core_axis_name="c", subcore_axis_name="s")
  sc_params = pltpu.CompilerParams(use_tc_tiling_on_sc=False)

  s1o = pl.kernel(
      _sc1_body,
      out_type=jax.ShapeDtypeStruct((_NC, _N_PAD, 48), F32),
      mesh=mesh,
      compiler_params=sc_params,
      scratch_types=[
          pltpu.VMEM((_GPT, 128), jnp.int32),
          pltpu.VMEM((_GPT, 128), jnp.int32),
          pltpu.VMEM((512, 48), F32),
          pltpu.VMEM((512, 48), F32),
          pltpu.VMEM_SHARED((_N_PAD, 48), F32),
          pltpu.SemaphoreType.DMA,
          pltpu.SemaphoreType.DMA,
      ],
  )(src_g, dst_s, v48, z48)

  x1, gsrc, gdst = pl.pallas_call(
      _tc2_body,
      out_shape=[jax.ShapeDtypeStruct((n, h), F32),
                 jax.ShapeDtypeStruct((_N_PAD, h), F32),
                 jax.ShapeDtypeStruct((_N_PAD, h), F32)],
  )(v48, r, s1o, a1, b1, bn1, be1)

  h1, s2o = pl.kernel(
      _sc2_body,
      out_type=[jax.ShapeDtypeStruct((_E_PAD, 32), F32),
                jax.ShapeDtypeStruct((_NC, _N_PAD, 32), F32)],
      mesh=mesh,
      compiler_params=sc_params,
      scratch_types=[
          pltpu.VMEM((_GPT, 128), jnp.int32),
          pltpu.VMEM((_GPT, 128), jnp.int32),
          pltpu.VMEM((512, 32), F32),
          pltpu.VMEM((512, 32), F32),
          pltpu.VMEM((512, 32), F32),
          pltpu.VMEM((512, 32), F32),
          pltpu.VMEM((512, 32), F32),
          pltpu.VMEM((512, 32), F32),
          pltpu.VMEM_SHARED((_N_PAD, 32), F32),
          pltpu.SemaphoreType.DMA,
          pltpu.SemaphoreType.DMA,
          pltpu.SemaphoreType.DMA,
      ],
  )(src_g, dst_s, gsrc, gdst, z32)

  tsrc, tdst = pl.pallas_call(
      _tc3_body,
      out_shape=[jax.ShapeDtypeStruct((n, h), F32)] * 2,
  )(x1, s2o, s1o, wn2a, wn2b, bn2, a2, b2, be2)

  tsum = pl.kernel(
      _sc3_body,
      out_type=jax.ShapeDtypeStruct((_E_PAD, 32), F32),
      mesh=mesh,
      compiler_params=sc_params,
      scratch_types=[
          pltpu.VMEM((_GPT, 128), jnp.int32),
          pltpu.VMEM((_GPT, 128), jnp.int32),
          pltpu.VMEM((512, 32), F32),
          pltpu.VMEM((512, 32), F32),
          pltpu.VMEM((512, 32), F32),
          pltpu.VMEM((512, 32), F32),
          pltpu.VMEM((512, 32), F32),
          pltpu.VMEM((512, 32), F32),
          pltpu.SemaphoreType.DMA,
          pltpu.SemaphoreType.DMA,
      ],
  )(src_g, dst_g, tsrc, tdst)

  grid = (nblk,)
  pred2, loss1 = pl.pallas_call(
      _tc4_body,
      grid=grid,
      in_specs=[
          pl.BlockSpec((eb, 32), lambda i: (i, 0)),
          pl.BlockSpec((eb, 32), lambda i: (i, 0)),
          pl.BlockSpec((1, 1, eb), lambda i: (i, 0, 0)),
          pl.BlockSpec((32, 32), lambda i: (0, 0)),
          pl.BlockSpec((1, 32), lambda i: (0, 0)),
          pl.BlockSpec(memory_space=pltpu.SMEM),
      ],
      out_specs=[
          pl.BlockSpec((1, 1, eb), lambda i: (i, 0, 0)),
          pl.BlockSpec(memory_space=pltpu.SMEM),
      ],
      out_shape=[jax.ShapeDtypeStruct((nblk, 1, eb), F32),
                 jax.ShapeDtypeStruct((1,), F32)],
      scratch_shapes=[pltpu.SMEM((8,), F32)],
  )(h1, tsum, et2, c2, wc_row, b_c)

  return pred2.reshape(e), loss1[0]
